# baseline (speedup 1.0000x reference)
import functools

import jax
import jax.numpy as jnp
from jax import lax
from jax.experimental import pallas as pl
from jax.experimental.pallas import tpu as pltpu
from jax.experimental.pallas import tpu_sc as plsc

N = 10000
D_FEAT = 256
COORD = 3
HID = 512
CW = 128
NCHUNK = HID // CW
NSUB = 16
N_PAD = 10112
RPT = N_PAD // NSUB
E = 160000
UE = 2 * E
EB = 184
NB = 110
UE_PAD = NSUB * NB * EB
RB = 2000
GRID = N // RB
POOL = 20
PB = RB // POOL


def _idx_base(s, j):
  return pl.multiple_of((s * NB + j) * EB, EB)


def _fetch_idx(idx_hbm, j_base, buf, sem):
  pltpu.async_copy(idx_hbm.at[pl.ds(j_base, EB)], buf, sem)


def _wait_fill(idx_hbm, buf, sem):
  pltpu.make_async_copy(idx_hbm.at[pl.ds(0, EB)], buf, sem).wait()


def _sc_deg_body(dsts, zeros128, ones128, deg,
                 dacc, didx0, didx1, ones_v, semd0, semd1):
  c = lax.axis_index("c")
  s = lax.axis_index("s")

  pltpu.sync_copy(ones128, ones_v)
  pltpu.sync_copy(zeros128, dacc.at[pl.ds(pl.multiple_of(s * RPT, 8), RPT)])
  _fetch_idx(dsts, _idx_base(s, 0), didx0, semd0)
  _fetch_idx(dsts, _idx_base(s, 1), didx1, semd1)
  plsc.subcore_barrier()

  @pl.loop(0, NB, step=2)
  def _(j):
    _wait_fill(dsts, didx0, semd0)
    pltpu.sync_copy(ones_v, dacc.at[didx0], add=True)

    @pl.when(j + 2 < NB)
    def _():
      _fetch_idx(dsts, _idx_base(s, j + 2), didx0, semd0)

    _wait_fill(dsts, didx1, semd1)
    pltpu.sync_copy(ones_v, dacc.at[didx1], add=True)

    @pl.when(j + 3 < NB)
    def _():
      _fetch_idx(dsts, _idx_base(s, j + 3), didx1, semd1)

  plsc.subcore_barrier()

  @pl.when(c == 0)
  def _():
    pltpu.sync_copy(dacc.at[pl.ds(pl.multiple_of(s * RPT, 8), RPT)],
                    deg.at[pl.ds(pl.multiple_of(s * RPT, 8), RPT)])


def _sc_agg_body(hc0, hc1, hc2, hc3, srcs, dsts, zeros128,
                 agg0, agg1, agg2, agg3,
                 acc, si0, si1, di0, di1, r0, r1,
                 ss0, ss1, sd0, sd1, sg0, sg1):
  sidx, didx = [si0, si1], [di0, di1]
  rows = [r0, r1]
  sems, semd, semg = [ss0, ss1], [sd0, sd1], [sg0, sg1]
  c = lax.axis_index("c")
  s = lax.axis_index("s")

  def fetch_slot(j, q):
    _fetch_idx(srcs, _idx_base(s, j), sidx[q], sems[q])
    _fetch_idx(dsts, _idx_base(s, j), didx[q], semd[q])

  def do_chunk(table, out):
    pltpu.sync_copy(zeros128, acc.at[pl.ds(pl.multiple_of(s * RPT, 8), RPT)])
    fetch_slot(0, 0)
    fetch_slot(1, 1)
    plsc.subcore_barrier()

    @pl.loop(0, NB, step=2)
    def _(j):
      _wait_fill(srcs, sidx[0], sems[0])
      g0 = pltpu.async_copy(table.at[sidx[0]], rows[0], semg[0])
      _wait_fill(srcs, sidx[1], sems[1])
      g1 = pltpu.async_copy(table.at[sidx[1]], rows[1], semg[1])

      g0.wait()
      _wait_fill(dsts, didx[0], semd[0])
      pltpu.sync_copy(rows[0], acc.at[didx[0]], add=True)

      @pl.when(j + 2 < NB)
      def _():
        fetch_slot(j + 2, 0)

      g1.wait()
      _wait_fill(dsts, didx[1], semd[1])
      pltpu.sync_copy(rows[1], acc.at[didx[1]], add=True)

      @pl.when(j + 3 < NB)
      def _():
        fetch_slot(j + 3, 1)

    plsc.subcore_barrier()
    pltpu.sync_copy(acc.at[pl.ds(pl.multiple_of(s * RPT, 8), RPT)],
                    out.at[pl.ds(pl.multiple_of(s * RPT, 8), RPT)])
    plsc.subcore_barrier()

  @pl.when(c == 0)
  def _chunks01():
    do_chunk(hc0, agg0)
    do_chunk(hc1, agg1)

  @pl.when(c == 1)
  def _chunks23():
    do_chunk(hc2, agg2)
    do_chunk(hc3, agg3)


@functools.cache
def _sc_deg():
  return pl.kernel(
      _sc_deg_body,
      out_type=jax.ShapeDtypeStruct((N_PAD, CW), jnp.float32),
      mesh=plsc.VectorSubcoreMesh(core_axis_name="c", subcore_axis_name="s"),
      scratch_types=[
          pltpu.VMEM_SHARED((N_PAD, CW), jnp.float32),
          pltpu.VMEM((EB,), jnp.int32),
          pltpu.VMEM((EB,), jnp.int32),
          pltpu.VMEM((EB, CW), jnp.float32),
          pltpu.SemaphoreType.DMA,
          pltpu.SemaphoreType.DMA,
      ],
  )


@functools.cache
def _sc_aggregate():
  return pl.kernel(
      _sc_agg_body,
      out_type=(
          jax.ShapeDtypeStruct((N_PAD, CW), jnp.float32),
          jax.ShapeDtypeStruct((N_PAD, CW), jnp.float32),
          jax.ShapeDtypeStruct((N_PAD, CW), jnp.float32),
          jax.ShapeDtypeStruct((N_PAD, CW), jnp.float32),
      ),
      mesh=plsc.VectorSubcoreMesh(core_axis_name="c", subcore_axis_name="s"),
      scratch_types=(
          [pltpu.VMEM_SHARED((N_PAD, CW), jnp.float32)]
          + [pltpu.VMEM((EB,), jnp.int32)] * 4
          + [pltpu.VMEM((EB, CW), jnp.float32)] * 2
          + [pltpu.SemaphoreType.DMA] * 6
      ),
  )


def _proj_body(feat_ref, w_ref, b_ref, out_ref):
  h = lax.dot_general(feat_ref[...], w_ref[...], (((1,), (1,)), ((), ())),
                      preferred_element_type=jnp.float32) + b_ref[...]
  for cc in range(NCHUNK):
    out_ref[cc] = h[:, cc * CW:(cc + 1) * CW]


def _node_proj(feat, w, b):
  return pl.pallas_call(
      _proj_body,
      grid=(GRID,),
      in_specs=[
          pl.BlockSpec((RB, 384), lambda i: (i, 0)),
          pl.BlockSpec((HID, 384), lambda i: (0, 0)),
          pl.BlockSpec((1, HID), lambda i: (0, 0)),
      ],
      out_specs=pl.BlockSpec((NCHUNK, RB, CW), lambda i: (0, i, 0)),
      out_shape=jax.ShapeDtypeStruct((NCHUNK, N, CW), jnp.float32),
  )(feat, w, b)


def _mp_math(hc_ref, a0, a1, a2, a3, deg_ref, w_ref, b_ref):
  inv = 1.0 / jnp.maximum(deg_ref[:, 0:1], 1.0)
  m = b_ref[...]
  for cc, a in enumerate((a0, a1, a2, a3)):
    m = m + lax.dot_general(a[...] * inv, w_ref[:, cc * CW:(cc + 1) * CW],
                            (((1,), (1,)), ((), ())),
                            preferred_element_type=jnp.float32)
  g = 0.5 * m * (1.0 + lax.erf(m * (2.0 ** -0.5)))
  return [hc_ref[cc] + g[:, cc * CW:(cc + 1) * CW] for cc in range(NCHUNK)]


def _step_body(hc_ref, a0, a1, a2, a3, deg_ref, w_ref, b_ref, out_ref):
  h_new = _mp_math(hc_ref, a0, a1, a2, a3, deg_ref, w_ref, b_ref)
  for cc in range(NCHUNK):
    out_ref[cc] = h_new[cc]


_AGG_SPEC = pl.BlockSpec((RB, CW), lambda i: (i, 0))


def _mp_step(hc, aggs, deg, w, b):
  return pl.pallas_call(
      _step_body,
      grid=(GRID,),
      in_specs=[
          pl.BlockSpec((NCHUNK, RB, CW), lambda i: (0, i, 0)),
          _AGG_SPEC, _AGG_SPEC, _AGG_SPEC, _AGG_SPEC,
          pl.BlockSpec((RB, CW), lambda i: (i, 0)),
          pl.BlockSpec((HID, HID), lambda i: (0, 0)),
          pl.BlockSpec((1, HID), lambda i: (0, 0)),
      ],
      out_specs=pl.BlockSpec((NCHUNK, RB, CW), lambda i: (0, i, 0)),
      out_shape=jax.ShapeDtypeStruct((NCHUNK, N, CW), jnp.float32),
  )(hc, *aggs, deg, w, b)


def _final_body(hc_ref, a0, a1, a2, a3, deg_ref, w_ref, b_ref, p_ref, out_ref):
  h_new = _mp_math(hc_ref, a0, a1, a2, a3, deg_ref, w_ref, b_ref)
  h_full = jnp.concatenate(h_new, axis=1)
  out_ref[0] = lax.dot_general(p_ref[...], h_full, (((1,), (0,)), ((), ())),
                               preferred_element_type=jnp.float32)


def _final_step(hc, aggs, deg, w, b, pool_mat):
  return pl.pallas_call(
      _final_body,
      grid=(GRID,),
      in_specs=[
          pl.BlockSpec((NCHUNK, RB, CW), lambda i: (0, i, 0)),
          _AGG_SPEC, _AGG_SPEC, _AGG_SPEC, _AGG_SPEC,
          pl.BlockSpec((RB, CW), lambda i: (i, 0)),
          pl.BlockSpec((HID, HID), lambda i: (0, 0)),
          pl.BlockSpec((1, HID), lambda i: (0, 0)),
          pl.BlockSpec((PB, RB), lambda i: (0, 0)),
      ],
      out_specs=pl.BlockSpec((1, PB, HID), lambda i: (i, 0, 0)),
      out_shape=jax.ShapeDtypeStruct((GRID, PB, HID), jnp.float32),
  )(hc, *aggs, deg, w, b, pool_mat)


def kernel(fields, coords, connect, W_node, b_node, W0, b0, W1, b1, W2, b2):
  f32 = jnp.float32
  feat = jnp.concatenate(
      [fields[0], coords[0],
       jnp.zeros((N, 384 - D_FEAT - COORD), f32)], axis=1)
  w_pad = jnp.concatenate(
      [W_node, jnp.zeros((HID, 384 - D_FEAT - COORD), f32)], axis=1)

  src = connect[:, 0].astype(jnp.int32)
  dst = connect[:, 1].astype(jnp.int32)
  pad_n = UE_PAD - UE
  pad_src = jnp.zeros((pad_n,), jnp.int32)
  pad_dst = N + (jnp.arange(pad_n, dtype=jnp.int32) % (N_PAD - N))
  srcs = jnp.concatenate([src, dst, pad_src])
  dsts = jnp.concatenate([dst, src, pad_dst])

  zeros128 = jnp.zeros((RPT, CW), f32)
  ones128 = jnp.ones((EB, CW), f32)
  pool_mat = jnp.kron(jnp.eye(PB, dtype=f32), jnp.full((1, POOL), 1.0 / POOL, f32))

  hc = _node_proj(feat, w_pad, b_node[None, :])
  deg = _sc_deg()(dsts, zeros128, ones128)
  for step, (w, b) in enumerate(((W0, b0), (W1, b1), (W2, b2))):
    aggs = _sc_aggregate()(hc[0], hc[1], hc[2], hc[3], srcs, dsts, zeros128)
    if step < 2:
      hc = _mp_step(hc, aggs, deg, w, b[None, :])
    else:
      pooled = _final_step(hc, aggs, deg, w, b[None, :], pool_mat)
      pooled = pooled.reshape(N // POOL, HID)
  out = jnp.concatenate([pooled, pooled[:12]], axis=0)
  return out[None, :, :]


# speedup vs baseline: 1.1697x; 1.1697x over previous
import functools

import jax
import jax.numpy as jnp
from jax import lax
from jax.experimental import pallas as pl
from jax.experimental.pallas import tpu as pltpu
from jax.experimental.pallas import tpu_sc as plsc

N = 10000
D_FEAT = 256
COORD = 3
HID = 512
CW = 128
NCHUNK = HID // CW
NSUB = 16
N_PAD = 10112
RPT = N_PAD // NSUB
E = 160000
UE = 2 * E
EB = 160
NB = 126
UE_PAD = NSUB * NB * EB
RB = 2000
GRID = N // RB
POOL = 20
PB = RB // POOL


def _idx_base(s, j):
  return pl.multiple_of((s * NB + j) * EB, EB)


def _fetch_idx(idx_hbm, j_base, buf, sem):
  pltpu.async_copy(idx_hbm.at[pl.ds(j_base, EB)], buf, sem)


def _wait_fill(idx_hbm, buf, sem):
  pltpu.make_async_copy(idx_hbm.at[pl.ds(0, EB)], buf, sem).wait()


def _sc_deg_body(dsts, zeros128, ones128, deg,
                 dacc, didx0, didx1, ones_v, semd0, semd1):
  c = lax.axis_index("c")
  s = lax.axis_index("s")

  pltpu.sync_copy(ones128, ones_v)
  pltpu.sync_copy(zeros128, dacc.at[pl.ds(pl.multiple_of(s * RPT, 8), RPT)])
  _fetch_idx(dsts, _idx_base(s, 0), didx0, semd0)
  _fetch_idx(dsts, _idx_base(s, 1), didx1, semd1)
  plsc.subcore_barrier()

  @pl.loop(0, NB, step=2)
  def _(j):
    _wait_fill(dsts, didx0, semd0)
    pltpu.sync_copy(ones_v, dacc.at[didx0], add=True)

    @pl.when(j + 2 < NB)
    def _():
      _fetch_idx(dsts, _idx_base(s, j + 2), didx0, semd0)

    _wait_fill(dsts, didx1, semd1)
    pltpu.sync_copy(ones_v, dacc.at[didx1], add=True)

    @pl.when(j + 3 < NB)
    def _():
      _fetch_idx(dsts, _idx_base(s, j + 3), didx1, semd1)

  plsc.subcore_barrier()

  @pl.when(c == 0)
  def _():
    pltpu.sync_copy(dacc.at[pl.ds(pl.multiple_of(s * RPT, 8), RPT)],
                    deg.at[pl.ds(pl.multiple_of(s * RPT, 8), RPT)])


def _sc_agg_body(hc0, hc1, hc2, hc3, srcs, dsts, zeros128,
                 agg0, agg1, agg2, agg3,
                 acc, si0, si1, di0, di1, r0, r1,
                 ss0, ss1, sd0, sd1, sg0, sg1, sc0, sc1):
  sidx, didx = [si0, si1], [di0, di1]
  rows = [r0, r1]
  sems, semd, semg = [ss0, ss1], [sd0, sd1], [sg0, sg1]
  semsc = [sc0, sc1]
  c = lax.axis_index("c")
  s = lax.axis_index("s")

  def do_chunk(table, out):
    pltpu.sync_copy(zeros128, acc.at[pl.ds(pl.multiple_of(s * RPT, 8), RPT)])
    _fetch_idx(srcs, _idx_base(s, 0), sidx[0], sems[0])
    _fetch_idx(srcs, _idx_base(s, 1), sidx[1], sems[1])
    _fetch_idx(dsts, _idx_base(s, 0), didx[0], semd[0])
    _fetch_idx(dsts, _idx_base(s, 1), didx[1], semd[1])
    plsc.subcore_barrier()

    @pl.loop(0, NB, step=2)
    def _(j):
      _wait_fill(srcs, sidx[0], sems[0])

      @pl.when(j >= 2)
      def _():
        pltpu.make_async_copy(rows[0], acc.at[didx[0]], semsc[0]).wait()
        _fetch_idx(dsts, _idx_base(s, j), didx[0], semd[0])

      g0 = pltpu.async_copy(table.at[sidx[0]], rows[0], semg[0])
      _wait_fill(srcs, sidx[1], sems[1])

      @pl.when(j >= 2)
      def _():
        pltpu.make_async_copy(rows[1], acc.at[didx[1]], semsc[1]).wait()
        _fetch_idx(dsts, _idx_base(s, j + 1), didx[1], semd[1])

      g1 = pltpu.async_copy(table.at[sidx[1]], rows[1], semg[1])

      g0.wait()

      @pl.when(j + 2 < NB)
      def _():
        _fetch_idx(srcs, _idx_base(s, j + 2), sidx[0], sems[0])

      _wait_fill(dsts, didx[0], semd[0])
      pltpu.async_copy(rows[0], acc.at[didx[0]], semsc[0], add=True)

      g1.wait()

      @pl.when(j + 3 < NB)
      def _():
        _fetch_idx(srcs, _idx_base(s, j + 3), sidx[1], sems[1])

      _wait_fill(dsts, didx[1], semd[1])
      pltpu.async_copy(rows[1], acc.at[didx[1]], semsc[1], add=True)

    pltpu.make_async_copy(rows[0], acc.at[didx[0]], semsc[0]).wait()
    pltpu.make_async_copy(rows[1], acc.at[didx[1]], semsc[1]).wait()
    plsc.subcore_barrier()
    pltpu.sync_copy(acc.at[pl.ds(pl.multiple_of(s * RPT, 8), RPT)],
                    out.at[pl.ds(pl.multiple_of(s * RPT, 8), RPT)])
    plsc.subcore_barrier()

  @pl.when(c == 0)
  def _chunks01():
    do_chunk(hc0, agg0)
    do_chunk(hc1, agg1)

  @pl.when(c == 1)
  def _chunks23():
    do_chunk(hc2, agg2)
    do_chunk(hc3, agg3)


@functools.cache
def _sc_deg():
  return pl.kernel(
      _sc_deg_body,
      out_type=jax.ShapeDtypeStruct((N_PAD, CW), jnp.float32),
      mesh=plsc.VectorSubcoreMesh(core_axis_name="c", subcore_axis_name="s"),
      scratch_types=[
          pltpu.VMEM_SHARED((N_PAD, CW), jnp.float32),
          pltpu.VMEM((EB,), jnp.int32),
          pltpu.VMEM((EB,), jnp.int32),
          pltpu.VMEM((EB, CW), jnp.float32),
          pltpu.SemaphoreType.DMA,
          pltpu.SemaphoreType.DMA,
      ],
  )


@functools.cache
def _sc_aggregate():
  return pl.kernel(
      _sc_agg_body,
      out_type=(
          jax.ShapeDtypeStruct((N_PAD, CW), jnp.float32),
          jax.ShapeDtypeStruct((N_PAD, CW), jnp.float32),
          jax.ShapeDtypeStruct((N_PAD, CW), jnp.float32),
          jax.ShapeDtypeStruct((N_PAD, CW), jnp.float32),
      ),
      mesh=plsc.VectorSubcoreMesh(core_axis_name="c", subcore_axis_name="s"),
      scratch_types=(
          [pltpu.VMEM_SHARED((N_PAD, CW), jnp.float32)]
          + [pltpu.VMEM((EB,), jnp.int32)] * 4
          + [pltpu.VMEM((EB, CW), jnp.float32)] * 2
          + [pltpu.SemaphoreType.DMA] * 8
      ),
  )


def _proj_body(feat_ref, w_ref, b_ref, out_ref):
  h = lax.dot_general(feat_ref[...], w_ref[...], (((1,), (1,)), ((), ())),
                      preferred_element_type=jnp.float32) + b_ref[...]
  for cc in range(NCHUNK):
    out_ref[cc] = h[:, cc * CW:(cc + 1) * CW]


def _node_proj(feat, w, b):
  return pl.pallas_call(
      _proj_body,
      grid=(GRID,),
      in_specs=[
          pl.BlockSpec((RB, 384), lambda i: (i, 0)),
          pl.BlockSpec((HID, 384), lambda i: (0, 0)),
          pl.BlockSpec((1, HID), lambda i: (0, 0)),
      ],
      out_specs=pl.BlockSpec((NCHUNK, RB, CW), lambda i: (0, i, 0)),
      out_shape=jax.ShapeDtypeStruct((NCHUNK, N, CW), jnp.float32),
  )(feat, w, b)


def _mp_math(hc_ref, a0, a1, a2, a3, deg_ref, w_ref, b_ref):
  inv = 1.0 / jnp.maximum(deg_ref[:, 0:1], 1.0)
  m = b_ref[...]
  for cc, a in enumerate((a0, a1, a2, a3)):
    m = m + lax.dot_general(a[...] * inv, w_ref[:, cc * CW:(cc + 1) * CW],
                            (((1,), (1,)), ((), ())),
                            preferred_element_type=jnp.float32)
  g = 0.5 * m * (1.0 + lax.erf(m * (2.0 ** -0.5)))
  return [hc_ref[cc] + g[:, cc * CW:(cc + 1) * CW] for cc in range(NCHUNK)]


def _step_body(hc_ref, a0, a1, a2, a3, deg_ref, w_ref, b_ref, out_ref):
  h_new = _mp_math(hc_ref, a0, a1, a2, a3, deg_ref, w_ref, b_ref)
  for cc in range(NCHUNK):
    out_ref[cc] = h_new[cc]


_AGG_SPEC = pl.BlockSpec((RB, CW), lambda i: (i, 0))


def _mp_step(hc, aggs, deg, w, b):
  return pl.pallas_call(
      _step_body,
      grid=(GRID,),
      in_specs=[
          pl.BlockSpec((NCHUNK, RB, CW), lambda i: (0, i, 0)),
          _AGG_SPEC, _AGG_SPEC, _AGG_SPEC, _AGG_SPEC,
          pl.BlockSpec((RB, CW), lambda i: (i, 0)),
          pl.BlockSpec((HID, HID), lambda i: (0, 0)),
          pl.BlockSpec((1, HID), lambda i: (0, 0)),
      ],
      out_specs=pl.BlockSpec((NCHUNK, RB, CW), lambda i: (0, i, 0)),
      out_shape=jax.ShapeDtypeStruct((NCHUNK, N, CW), jnp.float32),
  )(hc, *aggs, deg, w, b)


def _final_body(hc_ref, a0, a1, a2, a3, deg_ref, w_ref, b_ref, p_ref, out_ref):
  h_new = _mp_math(hc_ref, a0, a1, a2, a3, deg_ref, w_ref, b_ref)
  h_full = jnp.concatenate(h_new, axis=1)
  out_ref[0] = lax.dot_general(p_ref[...], h_full, (((1,), (0,)), ((), ())),
                               preferred_element_type=jnp.float32)


def _final_step(hc, aggs, deg, w, b, pool_mat):
  return pl.pallas_call(
      _final_body,
      grid=(GRID,),
      in_specs=[
          pl.BlockSpec((NCHUNK, RB, CW), lambda i: (0, i, 0)),
          _AGG_SPEC, _AGG_SPEC, _AGG_SPEC, _AGG_SPEC,
          pl.BlockSpec((RB, CW), lambda i: (i, 0)),
          pl.BlockSpec((HID, HID), lambda i: (0, 0)),
          pl.BlockSpec((1, HID), lambda i: (0, 0)),
          pl.BlockSpec((PB, RB), lambda i: (0, 0)),
      ],
      out_specs=pl.BlockSpec((1, PB, HID), lambda i: (i, 0, 0)),
      out_shape=jax.ShapeDtypeStruct((GRID, PB, HID), jnp.float32),
  )(hc, *aggs, deg, w, b, pool_mat)


def kernel(fields, coords, connect, W_node, b_node, W0, b0, W1, b1, W2, b2):
  f32 = jnp.float32
  feat = jnp.concatenate(
      [fields[0], coords[0],
       jnp.zeros((N, 384 - D_FEAT - COORD), f32)], axis=1)
  w_pad = jnp.concatenate(
      [W_node, jnp.zeros((HID, 384 - D_FEAT - COORD), f32)], axis=1)

  src = connect[:, 0].astype(jnp.int32)
  dst = connect[:, 1].astype(jnp.int32)
  pad_n = UE_PAD - UE
  pad_src = jnp.zeros((pad_n,), jnp.int32)
  pad_dst = N + (jnp.arange(pad_n, dtype=jnp.int32) % (N_PAD - N))
  srcs = jnp.concatenate([src, dst, pad_src])
  dsts = jnp.concatenate([dst, src, pad_dst])

  zeros128 = jnp.zeros((RPT, CW), f32)
  ones128 = jnp.ones((EB, CW), f32)
  pool_mat = jnp.kron(jnp.eye(PB, dtype=f32), jnp.full((1, POOL), 1.0 / POOL, f32))

  hc = _node_proj(feat, w_pad, b_node[None, :])
  deg = _sc_deg()(dsts, zeros128, ones128)
  for step, (w, b) in enumerate(((W0, b0), (W1, b1), (W2, b2))):
    aggs = _sc_aggregate()(hc[0], hc[1], hc[2], hc[3], srcs, dsts, zeros128)
    if step < 2:
      hc = _mp_step(hc, aggs, deg, w, b[None, :])
    else:
      pooled = _final_step(hc, aggs, deg, w, b[None, :], pool_mat)
      pooled = pooled.reshape(N // POOL, HID)
  out = jnp.concatenate([pooled, pooled[:12]], axis=0)
  return out[None, :, :]
